# baseline (speedup 1.0000x reference)
import jax
import jax.numpy as jnp
from jax import lax
from jax.experimental import pallas as pl
from jax.experimental.pallas import tpu as pltpu
from jax.experimental.pallas import tpu_sc as plsc

S = 256
MAX_LEN = 8000
ROW0 = MAX_LEN - 1 - (S - 1)
NC = 2
NS = 16
NW = NC * NS
HPW = S // NW
RB = 8
NB = S // RB
NBUF = 4
L = 16


def _sc_body(x_hbm, table_hbm, out_hbm, twin, xblk, sem_x, sem_o):
    cid = lax.axis_index("c")
    sid = lax.axis_index("s")
    wid = sid * NC + cid
    h0 = wid * HPW

    ct = lax.div(h0, 128) * 128
    hcol = lax.rem(h0, 128)
    lane = lax.iota(jnp.int32, L)

    def stage_table(tblk):
        pltpu.sync_copy(table_hbm.at[pl.ds(ROW0, 2 * S), pl.ds(ct, 128)],
                        tblk)
        for hl in range(HPW):
            hsplat = jnp.full((L,), 0, jnp.int32) + (hcol + hl)
            for cc in range(2 * S // L):
                rows = lane + cc * L
                twin[hl, pl.ds(cc * L, L)] = plsc.load_gather(
                    tblk, [rows, hsplat])

    pl.run_scoped(stage_table, pltpu.VMEM((2 * S, 128), jnp.float32))

    pltpu.async_copy(x_hbm.at[pl.ds(0, RB), :], xblk.at[0], sem_x.at[0])

    def main(sbuf):
        def iblock(k, carry):
            ib = k * RB
            p = lax.rem(k, NBUF)
            px = lax.rem(k, 2)

            @pl.when(k >= NBUF)
            def _():
                for hl in range(HPW):
                    pltpu.make_async_copy(
                        sbuf.at[p, hl],
                        out_hbm.at[h0 + hl, pl.ds(0, RB), :],
                        sem_o.at[p]).wait()

            pltpu.make_async_copy(x_hbm.at[pl.ds(ib, RB), :], xblk.at[px],
                                  sem_x.at[px]).wait()

            @pl.when(k + 1 < NB)
            def _():
                pltpu.async_copy(x_hbm.at[pl.ds(ib + RB, RB), :],
                                 xblk.at[1 - px], sem_x.at[1 - px])

            lane2 = lax.iota(jnp.int32, L)

            @plsc.parallel_loop(0, RB * (S // L), unroll=1)
            def _(t):
                il = lax.shift_right_logical(t, 4)
                off = lax.shift_left(lax.bitwise_and(t, S // L - 1), 4)
                base = (S - 1) - (ib + il)
                rows = base + off + lane2
                xv = xblk[px, il, pl.ds(off, L)]
                for hl in range(HPW):
                    tv = plsc.load_gather(
                        twin, [jnp.full((L,), 0, jnp.int32) + hl, rows])
                    sbuf[p, hl, il, pl.ds(off, L)] = xv + tv

            for hl in range(HPW):
                for ctile in range(2):
                    pltpu.async_copy(
                        sbuf.at[p, hl, :, pl.ds(ctile * 128, 128)],
                        out_hbm.at[h0 + hl, pl.ds(ib, RB),
                                   pl.ds(ctile * 128, 128)],
                        sem_o.at[p])
            return carry

        lax.fori_loop(0, NB, iblock, 0)

        for p in range(NBUF):
            for hl in range(HPW):
                pltpu.make_async_copy(
                    sbuf.at[p, hl],
                    out_hbm.at[h0 + hl, pl.ds(0, RB), :],
                    sem_o.at[p]).wait()

    pl.run_scoped(main, pltpu.VMEM((NBUF, HPW, RB, S), jnp.float32))


@jax.jit
def _sc_call(xf, table):
    mesh = plsc.VectorSubcoreMesh(core_axis_name="c", subcore_axis_name="s")
    return pl.kernel(
        _sc_body,
        out_type=jax.ShapeDtypeStruct((S, S, S), jnp.float32),
        mesh=mesh,
        scratch_types=[
            pltpu.VMEM((HPW, 2 * S), jnp.float32),
            pltpu.VMEM((2, RB, S), jnp.float32),
            pltpu.SemaphoreType.DMA((2,)),
            pltpu.SemaphoreType.DMA((NBUF,)),
        ],
        compiler_params=pltpu.CompilerParams(use_tc_tiling_on_sc=True,
                                             needs_layout_passes=False),
        name="rel_pos_bias_sc",
    )(xf, table)


def kernel(x, relative_position_bias_table):
    xf = x[0]
    out = _sc_call(xf, relative_position_bias_table)
    return out[None]


# speedup vs baseline: 1.0772x; 1.0772x over previous
import jax
import jax.numpy as jnp
from jax import lax
from jax.experimental import pallas as pl
from jax.experimental.pallas import tpu as pltpu
from jax.experimental.pallas import tpu_sc as plsc

S = 256
MAX_LEN = 8000
ROW0 = MAX_LEN - 1 - (S - 1)
NC = 2
NS = 16
NW = NC * NS
HPW = S // NW
RB = 16
NB = S // RB
NBUF = 2
L = 16


def _sc_body(x_hbm, table_hbm, out_hbm, twin, xblk, sem_x, sem_o):
    cid = lax.axis_index("c")
    sid = lax.axis_index("s")
    wid = sid * NC + cid
    h0 = wid * HPW

    ct = lax.div(h0, 128) * 128
    hcol = lax.rem(h0, 128)
    lane = lax.iota(jnp.int32, L)

    def stage_table(tblk):
        pltpu.sync_copy(table_hbm.at[pl.ds(ROW0, 2 * S), pl.ds(ct, 128)],
                        tblk)
        for hl in range(HPW):
            hsplat = jnp.full((L,), 0, jnp.int32) + (hcol + hl)
            for cc in range(2 * S // L):
                rows = lane + cc * L
                twin[hl, pl.ds(cc * L, L)] = plsc.load_gather(
                    tblk, [rows, hsplat])

    pl.run_scoped(stage_table, pltpu.VMEM((2 * S, 128), jnp.float32))

    pltpu.async_copy(x_hbm.at[pl.ds(0, RB), :], xblk.at[0], sem_x.at[0])

    def main(sbuf):
        def iblock(k, carry):
            ib = k * RB
            p = lax.rem(k, NBUF)
            px = lax.rem(k, 2)

            @pl.when(k >= NBUF)
            def _():
                for hl in range(HPW):
                    pltpu.make_async_copy(
                        sbuf.at[p, hl],
                        out_hbm.at[h0 + hl, pl.ds(0, RB), :],
                        sem_o.at[p]).wait()

            pltpu.make_async_copy(x_hbm.at[pl.ds(ib, RB), :], xblk.at[px],
                                  sem_x.at[px]).wait()

            @pl.when(k + 1 < NB)
            def _():
                pltpu.async_copy(x_hbm.at[pl.ds(ib + RB, RB), :],
                                 xblk.at[1 - px], sem_x.at[1 - px])

            lane2 = lax.iota(jnp.int32, L)

            @plsc.parallel_loop(0, RB * (S // L), unroll=2)
            def _(t):
                il = lax.shift_right_logical(t, 4)
                off = lax.shift_left(lax.bitwise_and(t, S // L - 1), 4)
                base = (S - 1) - (ib + il)
                rows = base + off + lane2
                xv = xblk[px, il, pl.ds(off, L)]
                for hl in range(HPW):
                    tv = plsc.load_gather(
                        twin, [jnp.full((L,), 0, jnp.int32) + hl, rows])
                    sbuf[p, hl, il, pl.ds(off, L)] = xv + tv

            for hl in range(HPW):
                for ctile in range(2):
                    pltpu.async_copy(
                        sbuf.at[p, hl, :, pl.ds(ctile * 128, 128)],
                        out_hbm.at[h0 + hl, pl.ds(ib, RB),
                                   pl.ds(ctile * 128, 128)],
                        sem_o.at[p])
            return carry

        lax.fori_loop(0, NB, iblock, 0)

        for p in range(NBUF):
            for hl in range(HPW):
                pltpu.make_async_copy(
                    sbuf.at[p, hl],
                    out_hbm.at[h0 + hl, pl.ds(0, RB), :],
                    sem_o.at[p]).wait()

    pl.run_scoped(main, pltpu.VMEM((NBUF, HPW, RB, S), jnp.float32))


@jax.jit
def _sc_call(xf, table):
    mesh = plsc.VectorSubcoreMesh(core_axis_name="c", subcore_axis_name="s")
    return pl.kernel(
        _sc_body,
        out_type=jax.ShapeDtypeStruct((S, S, S), jnp.float32),
        mesh=mesh,
        scratch_types=[
            pltpu.VMEM((HPW, 2 * S), jnp.float32),
            pltpu.VMEM((2, RB, S), jnp.float32),
            pltpu.SemaphoreType.DMA((2,)),
            pltpu.SemaphoreType.DMA((NBUF,)),
        ],
        compiler_params=pltpu.CompilerParams(use_tc_tiling_on_sc=True,
                                             needs_layout_passes=False),
        name="rel_pos_bias_sc",
    )(xf, table)


def kernel(x, relative_position_bias_table):
    xf = x[0]
    out = _sc_call(xf, relative_position_bias_table)
    return out[None]


# speedup vs baseline: 1.0999x; 1.0210x over previous
import jax
import jax.numpy as jnp
from jax import lax
from jax.experimental import pallas as pl
from jax.experimental.pallas import tpu as pltpu
from jax.experimental.pallas import tpu_sc as plsc

S = 256
MAX_LEN = 8000
ROW0 = MAX_LEN - 1 - (S - 1)
NC = 2
NS = 16
NW = NC * NS
HPW = S // NW
RB = 8
NB = S // RB
NBUF = 4
L = 16


def _sc_body(x_hbm, table_hbm, out_hbm, twin, xblk, sem_x, sem_o):
    cid = lax.axis_index("c")
    sid = lax.axis_index("s")
    wid = sid * NC + cid
    h0 = wid * HPW

    ct = lax.div(h0, 128) * 128
    hcol = lax.rem(h0, 128)
    lane = lax.iota(jnp.int32, L)

    def stage_table(tblk):
        pltpu.sync_copy(table_hbm.at[pl.ds(ROW0, 2 * S), pl.ds(ct, 128)],
                        tblk)
        for hl in range(HPW):
            hsplat = jnp.full((L,), 0, jnp.int32) + (hcol + hl)
            for cc in range(2 * S // L):
                rows = lane + cc * L
                twin[hl, pl.ds(cc * L, L)] = plsc.load_gather(
                    tblk, [rows, hsplat])

    pl.run_scoped(stage_table, pltpu.VMEM((2 * S, 128), jnp.float32))

    pltpu.async_copy(x_hbm.at[pl.ds(0, RB), :], xblk.at[0], sem_x.at[0])

    def main(sbuf):
        def iblock(k, carry):
            ib = k * RB
            p = lax.rem(k, NBUF)
            px = lax.rem(k, 2)

            @pl.when(k >= NBUF)
            def _():
                for hl in range(HPW):
                    pltpu.make_async_copy(
                        sbuf.at[p, hl],
                        out_hbm.at[h0 + hl, pl.ds(0, RB), :],
                        sem_o.at[p]).wait()

            pltpu.make_async_copy(x_hbm.at[pl.ds(ib, RB), :], xblk.at[px],
                                  sem_x.at[px]).wait()

            @pl.when(k + 1 < NB)
            def _():
                pltpu.async_copy(x_hbm.at[pl.ds(ib + RB, RB), :],
                                 xblk.at[1 - px], sem_x.at[1 - px])

            lane2 = lax.iota(jnp.int32, L)

            @plsc.parallel_loop(0, RB * (S // L), unroll=2)
            def _(t):
                il = lax.shift_right_logical(t, 4)
                off = lax.shift_left(lax.bitwise_and(t, S // L - 1), 4)
                base = (S - 1) - (ib + il)
                rows = base + off + lane2
                xv = xblk[px, il, pl.ds(off, L)]
                for hl in range(HPW):
                    tv = plsc.load_gather(
                        twin, [jnp.full((L,), 0, jnp.int32) + hl, rows])
                    sbuf[p, hl, il, pl.ds(off, L)] = xv + tv

            for hl in range(HPW):
                for ctile in range(2):
                    pltpu.async_copy(
                        sbuf.at[p, hl, :, pl.ds(ctile * 128, 128)],
                        out_hbm.at[h0 + hl, pl.ds(ib, RB),
                                   pl.ds(ctile * 128, 128)],
                        sem_o.at[p])
            return carry

        lax.fori_loop(0, NB, iblock, 0)

        for p in range(NBUF):
            for hl in range(HPW):
                pltpu.make_async_copy(
                    sbuf.at[p, hl],
                    out_hbm.at[h0 + hl, pl.ds(0, RB), :],
                    sem_o.at[p]).wait()

    pl.run_scoped(main, pltpu.VMEM((NBUF, HPW, RB, S), jnp.float32))


@jax.jit
def _sc_call(xf, table):
    mesh = plsc.VectorSubcoreMesh(core_axis_name="c", subcore_axis_name="s")
    return pl.kernel(
        _sc_body,
        out_type=jax.ShapeDtypeStruct((S, S, S), jnp.float32),
        mesh=mesh,
        scratch_types=[
            pltpu.VMEM((HPW, 2 * S), jnp.float32),
            pltpu.VMEM((2, RB, S), jnp.float32),
            pltpu.SemaphoreType.DMA((2,)),
            pltpu.SemaphoreType.DMA((NBUF,)),
        ],
        compiler_params=pltpu.CompilerParams(use_tc_tiling_on_sc=True,
                                             needs_layout_passes=False),
        name="rel_pos_bias_sc",
    )(xf, table)


def kernel(x, relative_position_bias_table):
    xf = x[0]
    out = _sc_call(xf, relative_position_bias_table)
    return out[None]
